# baseline (speedup 1.0000x reference)
import functools

import jax
import jax.numpy as jnp
from jax import lax
from jax.experimental import pallas as pl
from jax.experimental.pallas import tpu as pltpu
from jax.experimental.pallas import tpu_sc as plsc

N_ATOMS = 100000
NUM_TYPES = 64
LANES = 16
NUM_WORKERS = 32
CHUNK = 3200
HALF = CHUNK // 2
LAST_BASE = N_ATOMS - CHUNK

_mesh = plsc.VectorSubcoreMesh(core_axis_name="c", subcore_axis_name="s")


@functools.partial(
    pl.kernel,
    mesh=_mesh,
    out_type=(
        jax.ShapeDtypeStruct((N_ATOMS,), jnp.float32),
        jax.ShapeDtypeStruct((N_ATOMS,), jnp.float32),
    ),
    compiler_params=pltpu.CompilerParams(needs_layout_passes=False),
    scratch_types=[
        pltpu.VMEM((CHUNK,), jnp.int32),
        pltpu.VMEM((CHUNK,), jnp.float32),
        pltpu.VMEM((CHUNK,), jnp.float32),
        pltpu.VMEM((NUM_TYPES,), jnp.float32),
        pltpu.VMEM((NUM_TYPES,), jnp.float32),
        pltpu.SemaphoreType.DMA,
        pltpu.SemaphoreType.DMA,
        pltpu.SemaphoreType.DMA,
    ],
)
def _gather_tables_sc(t_hbm, scales_hbm, shifts_hbm, s_out_hbm, b_out_hbm,
                      idx_v, s_v, b_v, sc_v, sh_v, sem_a, sem_b, sem_out):
    wid = lax.axis_index("s") * 2 + lax.axis_index("c")
    base = jnp.minimum(wid * CHUNK, LAST_BASE)

    ct1 = pltpu.async_copy(scales_hbm, sc_v, sem_a)
    ct2 = pltpu.async_copy(shifts_hbm, sh_v, sem_a)
    ca = pltpu.async_copy(t_hbm.at[pl.ds(base, HALF)],
                          idx_v.at[pl.ds(0, HALF)], sem_a)
    cb = pltpu.async_copy(t_hbm.at[pl.ds(base + HALF, HALF)],
                          idx_v.at[pl.ds(HALF, HALF)], sem_b)
    ct1.wait()
    ct2.wait()
    ca.wait()

    @plsc.parallel_loop(0, HALF, LANES, unroll=8)
    def _(i):
        sl = pl.ds(i, LANES)
        idx = idx_v[sl]
        s_v[sl] = plsc.load_gather(sc_v, [idx])
        b_v[sl] = plsc.load_gather(sh_v, [idx])

    co1 = pltpu.async_copy(s_v.at[pl.ds(0, HALF)],
                           s_out_hbm.at[pl.ds(base, HALF)], sem_out)
    co2 = pltpu.async_copy(b_v.at[pl.ds(0, HALF)],
                           b_out_hbm.at[pl.ds(base, HALF)], sem_out)
    cb.wait()

    @plsc.parallel_loop(HALF, CHUNK, LANES, unroll=8)
    def _(i):
        sl = pl.ds(i, LANES)
        idx = idx_v[sl]
        s_v[sl] = plsc.load_gather(sc_v, [idx])
        b_v[sl] = plsc.load_gather(sh_v, [idx])

    co3 = pltpu.async_copy(s_v.at[pl.ds(HALF, HALF)],
                           s_out_hbm.at[pl.ds(base + HALF, HALF)], sem_out)
    co4 = pltpu.async_copy(b_v.at[pl.ds(HALF, HALF)],
                           b_out_hbm.at[pl.ds(base + HALF, HALF)], sem_out)
    co1.wait()
    co2.wait()
    co3.wait()
    co4.wait()


def kernel(atomic_energy, atom_types, scales, shifts):
    t = atom_types.reshape(-1).astype(jnp.int32)
    s, b = _gather_tables_sc(t, scales.astype(jnp.float32),
                             shifts.astype(jnp.float32))
    x = atomic_energy.astype(jnp.float32)
    return b[:, None] + s[:, None] * x


# speedup vs baseline: 1.0632x; 1.0632x over previous
import functools

import jax
import jax.numpy as jnp
from jax import lax
from jax.experimental import pallas as pl
from jax.experimental.pallas import tpu as pltpu
from jax.experimental.pallas import tpu_sc as plsc

N_ATOMS = 100000
NUM_TYPES = 64
LANES = 16
CHUNK = 6400
LAST_BASE = N_ATOMS - CHUNK

_mesh = plsc.VectorSubcoreMesh(core_axis_name="c", subcore_axis_name="s",
                               num_cores=1)


@functools.partial(
    pl.kernel,
    mesh=_mesh,
    out_type=jax.ShapeDtypeStruct((N_ATOMS,), jnp.float32),
    compiler_params=pltpu.CompilerParams(needs_layout_passes=False),
    scratch_types=[
        pltpu.VMEM((CHUNK,), jnp.int32),
        pltpu.VMEM((CHUNK,), jnp.float32),
        pltpu.VMEM((CHUNK,), jnp.float32),
        pltpu.VMEM((NUM_TYPES,), jnp.float32),
        pltpu.VMEM((NUM_TYPES,), jnp.float32),
        pltpu.SemaphoreType.DMA,
    ],
)
def _scale_shift_sc(x_hbm, t_hbm, scales_hbm, shifts_hbm, out_hbm,
                    idx_v, x_v, o_v, sc_v, sh_v, sem):
    wid = lax.axis_index("s")
    base = jnp.minimum(wid * CHUNK, LAST_BASE)

    c1 = pltpu.async_copy(t_hbm.at[pl.ds(base, CHUNK)], idx_v, sem)
    c2 = pltpu.async_copy(x_hbm.at[pl.ds(base, CHUNK)], x_v, sem)
    c3 = pltpu.async_copy(scales_hbm, sc_v, sem)
    c4 = pltpu.async_copy(shifts_hbm, sh_v, sem)
    c1.wait()
    c2.wait()
    c3.wait()
    c4.wait()

    @plsc.parallel_loop(0, CHUNK, LANES, unroll=8)
    def _(i):
        sl = pl.ds(i, LANES)
        idx = idx_v[sl]
        s = plsc.load_gather(sc_v, [idx])
        b = plsc.load_gather(sh_v, [idx])
        o_v[sl] = b + s * x_v[sl]

    pltpu.sync_copy(o_v, out_hbm.at[pl.ds(base, CHUNK)])


def kernel(atomic_energy, atom_types, scales, shifts):
    x = atomic_energy.reshape(-1).astype(jnp.float32)
    t = atom_types.reshape(-1).astype(jnp.int32)
    out = _scale_shift_sc(x, t, scales.astype(jnp.float32),
                          shifts.astype(jnp.float32))
    return out.reshape(-1, 1)
